# baseline (speedup 1.0000x reference)
import functools

import jax
import jax.numpy as jnp
from jax import lax
from jax.experimental import pallas as pl
from jax.experimental.pallas import tpu as pltpu
from jax.experimental.pallas import tpu_sc as plsc

N = 10000
N_PAD = 10240
E = 320000
F_IN = 128
HID = 256
C = 40
D = 48
K = 10

NC = 2
NS = 16
NW = NC * NS
EPW = E // NW
CH = 128
NCH = -(-EPW // CH)
EPW_PAD = NCH * CH
ROWS_PER_TILE = N_PAD // NS


def _mlp_body(x_ref, w1_ref, b1_ref, gamma_ref, beta_ref, w2_ref, b2_ref,
              out_ref):
    x = x_ref[...]
    h = jnp.dot(x, w1_ref[...], preferred_element_type=jnp.float32)
    h = h + b1_ref[...]
    mean = jnp.mean(h, axis=0, keepdims=True)
    var = jnp.mean(jnp.square(h - mean), axis=0, keepdims=True)
    h = (h - mean) * jax.lax.rsqrt(var + 1e-5) * gamma_ref[...] + beta_ref[...]
    h = jnp.maximum(h, 0.0)
    h2 = jnp.dot(h, w2_ref[...], preferred_element_type=jnp.float32)
    out_ref[...] = h2 + b2_ref[...]


def _mlp(x, W1, b1, gamma, beta, W2p, b2p):
    return pl.pallas_call(
        _mlp_body,
        out_shape=jax.ShapeDtypeStruct((N, D), jnp.float32),
    )(x, W1, b1.reshape(1, HID), gamma.reshape(1, HID), beta.reshape(1, HID),
      W2p, b2p.reshape(1, D))


def _round_body(cur_hbm, srcs_hbm, dsts_hbm, norms_hbm, zeros_hbm,
                parts_hbm, src_v, dst_v, norm_v, rows_v, acc_sh, gsem, ssem):
    cid = lax.axis_index("c")
    sid = lax.axis_index("s")
    wid = cid * NS + sid
    pltpu.sync_copy(srcs_hbm.at[wid], src_v)
    pltpu.sync_copy(dsts_hbm.at[wid], dst_v)
    pltpu.sync_copy(norms_hbm.at[wid], norm_v)
    row0 = sid * ROWS_PER_TILE
    pltpu.sync_copy(zeros_hbm, acc_sh.at[pl.ds(row0, ROWS_PER_TILE)])
    plsc.subcore_barrier()

    for b in range(5):
        pltpu.async_copy(cur_hbm.at[src_v.at[pl.ds(b * CH, CH)]], rows_v.at[b],
                         gsem.at[b])

    def chunk_body(j, carry):
        p = lax.rem(j, 6)

        @pl.when(j + 5 < NCH)
        def _start_next():
            q = lax.rem(j + 5, 6)

            @pl.when(j >= 1)
            def _drain_prev_scatter():
                pltpu.make_async_copy(
                    rows_v.at[q], acc_sh.at[dst_v.at[pl.ds((j - 1) * CH, CH)]],
                    ssem.at[q]
                ).wait()
            pltpu.async_copy(cur_hbm.at[src_v.at[pl.ds((j + 5) * CH, CH)]],
                             rows_v.at[q], gsem.at[q])

        pltpu.make_async_copy(cur_hbm.at[src_v.at[pl.ds(j * CH, CH)]],
                              rows_v.at[p], gsem.at[p]).wait()

        def grp_body(g, c2):
            nv = norm_v[pl.ds(j * CH + g * 16, 16)]
            for l in range(16):
                ns = nv[l]
                i = g * 16 + l
                for c in range(D // 16):
                    sl = pl.ds(c * 16, 16)
                    rows_v[p, i, sl] = rows_v[p, i, sl] * ns
            return c2

        lax.fori_loop(0, CH // 16, grp_body, 0)
        pltpu.async_copy(rows_v.at[p], acc_sh.at[dst_v.at[pl.ds(j * CH, CH)]],
                         ssem.at[p], add=True)
        return carry

    lax.fori_loop(0, NCH, chunk_body, 0)
    for t in range(NCH - 6, NCH):
        pltpu.make_async_copy(rows_v.at[t % 6],
                              acc_sh.at[dst_v.at[pl.ds(t * CH, CH)]],
                              ssem.at[t % 6]).wait()
    plsc.subcore_barrier()
    pltpu.sync_copy(acc_sh.at[pl.ds(row0, ROWS_PER_TILE)],
                    parts_hbm.at[cid, pl.ds(row0, ROWS_PER_TILE)])


_sc_round = functools.partial(
    pl.kernel,
    out_type=jax.ShapeDtypeStruct((NC, N_PAD, D), jnp.float32),
    mesh=plsc.VectorSubcoreMesh(core_axis_name="c", subcore_axis_name="s"),
    compiler_params=pltpu.CompilerParams(use_tc_tiling_on_sc=False),
    scratch_types=[
        pltpu.VMEM((EPW_PAD,), jnp.int32),
        pltpu.VMEM((EPW_PAD,), jnp.int32),
        pltpu.VMEM((EPW_PAD,), jnp.float32),
        pltpu.VMEM((6, CH, D), jnp.float32),
        pltpu.VMEM_SHARED((N_PAD, D), jnp.float32),
        pltpu.SemaphoreType.DMA((6,)),
        pltpu.SemaphoreType.DMA((6,)),
    ],
)(_round_body)


def _merge_body(p_ref, o_ref):
    o_ref[...] = p_ref[0] + p_ref[1]


def _merge(parts):
    grid = 10
    rows = N_PAD // grid
    return pl.pallas_call(
        _merge_body,
        grid=(grid,),
        in_specs=[pl.BlockSpec((NC, rows, D), lambda i: (0, i, 0))],
        out_specs=pl.BlockSpec((rows, D), lambda i: (i, 0)),
        out_shape=jax.ShapeDtypeStruct((N_PAD, D), jnp.float32),
    )(parts)


def _combine_body(wp_ref, bp_ref, *refs):
    pps_refs = refs[:K + 1]
    out_ref = refs[K + 1]
    wp = wp_ref[...]
    bp = bp_ref[0, 0]
    rows = pps_refs[0].shape[0]
    acc = jnp.zeros((rows, D), jnp.float32)
    for k in range(K + 1):
        p = pps_refs[k][...]
        score = jnp.sum(p * wp, axis=1, keepdims=True) + bp
        retain = jax.nn.sigmoid(score)
        acc = acc + retain * p
    lane = lax.broadcasted_iota(jnp.int32, (rows, D), 1)
    neg = jnp.float32(-1e30)
    am = jnp.where(lane < C, acc, neg)
    m = jnp.max(am, axis=1, keepdims=True)
    lse = jnp.log(jnp.sum(jnp.exp(am - m), axis=1, keepdims=True))
    z = acc - m - lse
    out_ref[...] = z[:, :C]


def _combine(pps_list, Wp_pad, bp):
    grid = 10
    rows = N_PAD // grid
    return pl.pallas_call(
        _combine_body,
        grid=(grid,),
        in_specs=[pl.BlockSpec((1, D), lambda i: (0, 0)),
                  pl.BlockSpec((1, 1), lambda i: (0, 0))] +
                 [pl.BlockSpec((rows, D), lambda i: (i, 0))] * (K + 1),
        out_specs=pl.BlockSpec((rows, C), lambda i: (i, 0)),
        out_shape=jax.ShapeDtypeStruct((N_PAD, C), jnp.float32),
    )(Wp_pad, bp.reshape(1, 1), *pps_list)


def kernel(x, norm, W1, b1, gamma, beta, W2, b2, Wp, bp, edge_index):
    W2p = jnp.pad(W2, ((0, 0), (0, D - C)))
    b2p = jnp.pad(b2, (0, D - C))
    Wp_pad = jnp.pad(Wp.reshape(1, C), ((0, 0), (0, D - C)))
    h = jnp.pad(_mlp(x, W1, b1, gamma, beta, W2p, b2p),
                ((0, N_PAD - N), (0, 0)))

    pad = EPW_PAD - EPW
    srcs = jnp.pad(edge_index[0].reshape(NW, EPW), ((0, 0), (0, pad)))
    dsts = jnp.pad(edge_index[1].reshape(NW, EPW), ((0, 0), (0, pad)))
    norms = jnp.pad(norm.reshape(NW, EPW), ((0, 0), (0, pad)))
    zeros = jnp.zeros((ROWS_PER_TILE, D), jnp.float32)

    preds = [h]
    cur = h
    for _ in range(K):
        parts = _sc_round(cur, srcs, dsts, norms, zeros)
        cur = _merge(parts)
        preds.append(cur)
    return _combine(preds, Wp_pad, bp)[:N]


# speedup vs baseline: 1.7650x; 1.7650x over previous
import functools

import jax
import jax.numpy as jnp
from jax import lax
from jax.experimental import pallas as pl
from jax.experimental.pallas import tpu as pltpu
from jax.experimental.pallas import tpu_sc as plsc

N = 10000
N_PAD = 10240
E = 320000
F_IN = 128
HID = 256
C = 40
D = 48
K = 10

NC = 2
NS = 16
NW = NC * NS
EPW = E // NW
CH = 128
NCH = -(-EPW // CH)
EPW_PAD = NCH * CH
ROWS_PER_TILE = N_PAD // NS


def _mlp_body(x_ref, w1_ref, b1_ref, gamma_ref, beta_ref, w2_ref, b2_ref,
              out_ref):
    x = x_ref[...]
    h = jnp.dot(x, w1_ref[...], preferred_element_type=jnp.float32)
    h = h + b1_ref[...]
    mean = jnp.mean(h, axis=0, keepdims=True)
    var = jnp.mean(jnp.square(h - mean), axis=0, keepdims=True)
    h = (h - mean) * jax.lax.rsqrt(var + 1e-5) * gamma_ref[...] + beta_ref[...]
    h = jnp.maximum(h, 0.0)
    h2 = jnp.dot(h, w2_ref[...], preferred_element_type=jnp.float32)
    out_ref[...] = h2 + b2_ref[...]


def _mlp(x, W1, b1, gamma, beta, W2p, b2p):
    return pl.pallas_call(
        _mlp_body,
        out_shape=jax.ShapeDtypeStruct((N, D), jnp.float32),
    )(x, W1, b1.reshape(1, HID), gamma.reshape(1, HID), beta.reshape(1, HID),
      W2p, b2p.reshape(1, D))


def _round_body(cur_hbm, srcs_hbm, dsts_hbm, norms_hbm, zeros_hbm,
                parts_hbm, src_v, dst_v, norm_v, rows_v, acc_sh, gsem, ssem):
    cid = lax.axis_index("c")
    sid = lax.axis_index("s")
    wid = cid * NS + sid
    pltpu.sync_copy(srcs_hbm.at[wid], src_v)
    pltpu.sync_copy(dsts_hbm.at[wid], dst_v)
    pltpu.sync_copy(norms_hbm.at[wid], norm_v)
    row0 = sid * ROWS_PER_TILE
    pltpu.sync_copy(zeros_hbm, acc_sh.at[pl.ds(row0, ROWS_PER_TILE)])
    plsc.subcore_barrier()

    for b in range(3):
        pltpu.async_copy(cur_hbm.at[src_v.at[pl.ds(b * CH, CH)]], rows_v.at[b],
                         gsem.at[b])

    def chunk_body(j, carry):
        p = lax.rem(j, 4)

        @pl.when(j + 3 < NCH)
        def _start_next():
            q = lax.rem(j + 3, 4)

            @pl.when(j >= 1)
            def _drain_prev_scatter():
                pltpu.make_async_copy(
                    rows_v.at[q], acc_sh.at[dst_v.at[pl.ds((j - 1) * CH, CH)]],
                    ssem.at[q]
                ).wait()
            pltpu.async_copy(cur_hbm.at[src_v.at[pl.ds((j + 3) * CH, CH)]],
                             rows_v.at[q], gsem.at[q])

        pltpu.make_async_copy(cur_hbm.at[src_v.at[pl.ds(j * CH, CH)]],
                              rows_v.at[p], gsem.at[p]).wait()

        def grp_body(g, c2):
            nv = norm_v[pl.ds(j * CH + g * 16, 16)]
            for l in range(16):
                ns = nv[l]
                i = g * 16 + l
                for c in range(D // 16):
                    sl = pl.ds(c * 16, 16)
                    rows_v[p, i, sl] = rows_v[p, i, sl] * ns
            return c2

        lax.fori_loop(0, CH // 16, grp_body, 0)
        pltpu.async_copy(rows_v.at[p], acc_sh.at[dst_v.at[pl.ds(j * CH, CH)]],
                         ssem.at[p], add=True)
        return carry

    lax.fori_loop(0, NCH, chunk_body, 0)
    for t in range(NCH - 4, NCH):
        pltpu.make_async_copy(rows_v.at[t % 4],
                              acc_sh.at[dst_v.at[pl.ds(t * CH, CH)]],
                              ssem.at[t % 4]).wait()
    plsc.subcore_barrier()
    pltpu.sync_copy(acc_sh.at[pl.ds(row0, ROWS_PER_TILE)],
                    parts_hbm.at[cid, pl.ds(row0, ROWS_PER_TILE)])


_sc_round = functools.partial(
    pl.kernel,
    out_type=jax.ShapeDtypeStruct((NC, N_PAD, D), jnp.float32),
    mesh=plsc.VectorSubcoreMesh(core_axis_name="c", subcore_axis_name="s"),
    compiler_params=pltpu.CompilerParams(use_tc_tiling_on_sc=False),
    scratch_types=[
        pltpu.VMEM((EPW_PAD,), jnp.int32),
        pltpu.VMEM((EPW_PAD,), jnp.int32),
        pltpu.VMEM((EPW_PAD,), jnp.float32),
        pltpu.VMEM((4, CH, D), jnp.float32),
        pltpu.VMEM_SHARED((N_PAD, D), jnp.float32),
        pltpu.SemaphoreType.DMA((4,)),
        pltpu.SemaphoreType.DMA((4,)),
    ],
)(_round_body)


def _merge_body(p_ref, o_ref):
    o_ref[...] = p_ref[0] + p_ref[1]


def _merge(parts):
    grid = 10
    rows = N_PAD // grid
    return pl.pallas_call(
        _merge_body,
        grid=(grid,),
        in_specs=[pl.BlockSpec((NC, rows, D), lambda i: (0, i, 0))],
        out_specs=pl.BlockSpec((rows, D), lambda i: (i, 0)),
        out_shape=jax.ShapeDtypeStruct((N_PAD, D), jnp.float32),
    )(parts)


def _combine_body(wp_ref, bp_ref, *refs):
    pps_refs = refs[:K + 1]
    out_ref = refs[K + 1]
    wp = wp_ref[...]
    bp = bp_ref[0, 0]
    rows = pps_refs[0].shape[0]
    acc = jnp.zeros((rows, D), jnp.float32)
    for k in range(K + 1):
        p = pps_refs[k][...]
        score = jnp.sum(p * wp, axis=1, keepdims=True) + bp
        retain = jax.nn.sigmoid(score)
        acc = acc + retain * p
    lane = lax.broadcasted_iota(jnp.int32, (rows, D), 1)
    neg = jnp.float32(-1e30)
    am = jnp.where(lane < C, acc, neg)
    m = jnp.max(am, axis=1, keepdims=True)
    lse = jnp.log(jnp.sum(jnp.exp(am - m), axis=1, keepdims=True))
    z = acc - m - lse
    out_ref[...] = z[:, :C]


def _combine(pps_list, Wp_pad, bp):
    grid = 10
    rows = N_PAD // grid
    return pl.pallas_call(
        _combine_body,
        grid=(grid,),
        in_specs=[pl.BlockSpec((1, D), lambda i: (0, 0)),
                  pl.BlockSpec((1, 1), lambda i: (0, 0))] +
                 [pl.BlockSpec((rows, D), lambda i: (i, 0))] * (K + 1),
        out_specs=pl.BlockSpec((rows, C), lambda i: (i, 0)),
        out_shape=jax.ShapeDtypeStruct((N_PAD, C), jnp.float32),
    )(Wp_pad, bp.reshape(1, 1), *pps_list)


def kernel(x, norm, W1, b1, gamma, beta, W2, b2, Wp, bp, edge_index):
    W2p = jnp.pad(W2, ((0, 0), (0, D - C)))
    b2p = jnp.pad(b2, (0, D - C))
    Wp_pad = jnp.pad(Wp.reshape(1, C), ((0, 0), (0, D - C)))
    h = jnp.pad(_mlp(x, W1, b1, gamma, beta, W2p, b2p),
                ((0, N_PAD - N), (0, 0)))

    pad = EPW_PAD - EPW
    srcs = jnp.pad(edge_index[0].reshape(NW, EPW), ((0, 0), (0, pad)))
    dsts = jnp.pad(edge_index[1].reshape(NW, EPW), ((0, 0), (0, pad)))
    norms = jnp.pad(norm.reshape(NW, EPW), ((0, 0), (0, pad)))
    zeros = jnp.zeros((ROWS_PER_TILE, D), jnp.float32)

    preds = [h]
    cur = h
    for _ in range(K):
        parts = _sc_round(cur, srcs, dsts, norms, zeros)
        cur = _merge(parts)
        preds.append(cur)
    return _combine(preds, Wp_pad, bp)[:N]
